# baseline (speedup 1.0000x reference)
import functools

import jax
import jax.numpy as jnp
from jax import lax
from jax.experimental import pallas as pl
from jax.experimental.pallas import tpu as pltpu
from jax.experimental.pallas import tpu_sc as plsc

_ROWS = 8192
_COLS = 768
_NUM_WORKERS = 32
_ROWS_PER_WORKER = _ROWS // _NUM_WORKERS
_CHUNK_ROWS = 64
_CHUNKS = _ROWS_PER_WORKER // _CHUNK_ROWS
_SLOTS = 2

_mesh = plsc.VectorSubcoreMesh(core_axis_name="c", subcore_axis_name="s")


@functools.partial(
    pl.kernel,
    mesh=_mesh,
    out_type=jax.ShapeDtypeStruct((_ROWS, _COLS), jnp.float32),
    scratch_types=[
        pltpu.VMEM((_SLOTS, _CHUNK_ROWS, _COLS), jnp.float32),
        pltpu.SemaphoreType.DMA((_SLOTS,)),
        pltpu.SemaphoreType.DMA((_SLOTS,)),
    ],
)
def _sc_copy(x_hbm, o_hbm, buf, in_sems, out_sems):
    wid = lax.axis_index("s") * 2 + lax.axis_index("c")
    base = wid * _ROWS_PER_WORKER

    def in_copy(c):
        return pltpu.make_async_copy(
            x_hbm.at[pl.ds(base + c * _CHUNK_ROWS, _CHUNK_ROWS), :],
            buf.at[c % _SLOTS],
            in_sems.at[c % _SLOTS],
        )

    def out_copy(c):
        return pltpu.make_async_copy(
            buf.at[c % _SLOTS],
            o_hbm.at[pl.ds(base + c * _CHUNK_ROWS, _CHUNK_ROWS), :],
            out_sems.at[c % _SLOTS],
        )

    for c in range(_SLOTS):
        in_copy(c).start()
    for c in range(_CHUNKS):
        if c >= 1 and c - 1 + _SLOTS < _CHUNKS:
            out_copy(c - 1).wait()
            in_copy(c - 1 + _SLOTS).start()
        in_copy(c).wait()
        out_copy(c).start()
    for c in range(max(_CHUNKS - _SLOTS, 0), _CHUNKS):
        out_copy(c).wait()


def kernel(prototypes):
    return _sc_copy(prototypes)
